# baseline (speedup 1.0000x reference)
import dataclasses
import functools

import jax
import jax.numpy as jnp
from jax import lax
from jax.experimental import pallas as pl
from jax.experimental.pallas import tpu as pltpu
from jax.experimental.pallas import tpu_sc as plsc

B = 512
NU = 1000
NI = 1000
D = 64
H = D
FU = 4
FI = 4
IN = D + FU + D + FI

RB = 50

SC_NC = 2
SC_NS = 16
SC_NW = SC_NC * SC_NS
BPW = B // SC_NW
HALF = BPW // 2


def _sc_gather(t2_u, t2_i, idx_u, idx_i):
    mesh = plsc.VectorSubcoreMesh(core_axis_name="c", subcore_axis_name="s")
    cp = pltpu.CompilerParams()
    if "needs_layout_passes" in pltpu.CompilerParams.__dataclass_fields__:
        cp = dataclasses.replace(cp, needs_layout_passes=False)

    @functools.partial(
        pl.kernel,
        mesh=mesh,
        out_type=(jax.ShapeDtypeStruct((B, D), jnp.float32),
                  jax.ShapeDtypeStruct((B, D), jnp.float32)),
        scratch_types=[
            pltpu.VMEM((BPW * D,), jnp.int32),
            pltpu.VMEM((HALF * D, 128), jnp.float32),
            pltpu.VMEM((BPW, D), jnp.float32),
            pltpu.SemaphoreType.DMA,
        ],
        compiler_params=cp,
    )
    def gather_kernel(tu, ti, ixu, ixi, ou, oi, idx_v, rows_v, stage, sem):
        wid = lax.axis_index("s") * SC_NC + lax.axis_index("c")
        base_b = wid * BPW
        iota16 = lax.broadcasted_iota(jnp.int32, (16,), 0)
        for t2, ixh, outh in ((tu, ixu, ou), (ti, ixi, oi)):
            pltpu.sync_copy(ixh.at[pl.ds(base_b * D, BPW * D)], idx_v)
            for half in range(2):
                pltpu.async_copy(
                    t2.at[idx_v.at[pl.ds(half * HALF * D, HALF * D)]],
                    rows_v, sem).wait()
                for bl in range(HALF):
                    bg = base_b + half * HALF + bl
                    lane_vec = jnp.broadcast_to(
                        lax.rem(bg, 128), (16,)).astype(jnp.int32)
                    for c in range(D // 16):
                        ridx = bl * D + 16 * c + iota16
                        vals = plsc.load_gather(rows_v, [ridx, lane_vec])
                        stage[half * HALF + bl, pl.ds(16 * c, 16)] = vals
            pltpu.sync_copy(stage, outh.at[pl.ds(base_b, BPW)])

    return gather_kernel(t2_u, t2_i, idx_u, idx_i)


def _gru_body(ue, ie, uf, if_,
              we1_u, wf1_u, we2_u, wf2_u, bih_u, bhh_u,
              we1_i, wf1_i, we2_i, wf2_i, bih_i, bhh_i,
              uo, io):
    ue_v, ie_v, uf_v, if_v = ue[...], ie[...], uf[...], if_[...]
    for emb, feat, emb2, feat2, we1, wf1, we2, wf2, bih, bhh, out in (
        (ue_v, uf_v, ie_v, if_v,
         we1_u, wf1_u, we2_u, wf2_u, bih_u, bhh_u, uo),
        (ie_v, if_v, ue_v, uf_v,
         we1_i, wf1_i, we2_i, wf2_i, bih_i, bhh_i, io),
    ):
        gi = lax.dot_general(we1[...], emb, (((1,), (1,)), ((), ())),
                             preferred_element_type=jnp.float32)
        gi = gi + lax.dot_general(wf1[...], feat, (((1,), (0,)), ((), ())),
                                  preferred_element_type=jnp.float32)
        gi = gi + lax.dot_general(we2[...], emb2, (((1,), (1,)), ((), ())),
                                  preferred_element_type=jnp.float32)
        gi = gi + lax.dot_general(wf2[...], feat2, (((1,), (0,)), ((), ())),
                                  preferred_element_type=jnp.float32)
        gi = gi + bih[...]
        bhh_v = bhh[...]
        r = jax.nn.sigmoid(gi[:H] + bhh_v[:H])
        z = jax.nn.sigmoid(gi[H:2 * H] + bhh_v[H:2 * H])
        n = jnp.tanh(gi[2 * H:] + r * bhh_v[2 * H:])
        h = (1.0 - z) * n
        nrm = jnp.sqrt(jnp.sum(h * h, axis=0, keepdims=True))
        out[...] = h / jnp.maximum(nrm, 1e-12)


def _gru(ue_bm, ie_bm, uf_t, if_t, wsplits_u, bih_u, bhh_u,
         wsplits_i, bih_i, bhh_i):
    return pl.pallas_call(
        _gru_body,
        out_shape=(jax.ShapeDtypeStruct((H, B), jnp.float32),
                   jax.ShapeDtypeStruct((H, B), jnp.float32)),
    )(ue_bm, ie_bm, uf_t, if_t, *wsplits_u, bih_u, bhh_u,
      *wsplits_i, bih_i, bhh_i)


def _scatter_body(ids_u_ref, ids_i_ref, emb_u_ref, emb_i_ref,
                  tu_ref, ti_ref, uo_ref, io_ref):
    r0 = pl.program_id(0) * RB
    ids_u = ids_u_ref[...]
    ids_i = ids_i_ref[...]
    emb_u = emb_u_ref[...]
    emb_i = emb_i_ref[...]
    for rr in range(RB):
        uo_ref[rr] = jnp.where(ids_u == r0 + rr, emb_u, tu_ref[rr])
        io_ref[rr] = jnp.where(ids_i == r0 + rr, emb_i, ti_ref[rr])


def _copy_scatter(t_u, t_i, emb_u_t, emb_i_t, ids_u, ids_i):
    c0 = lambda i: (0, 0)
    blk = lambda i: (i, 0, 0)
    return pl.pallas_call(
        _scatter_body,
        grid=(NU // RB,),
        in_specs=[
            pl.BlockSpec((1, B), c0),
            pl.BlockSpec((1, B), c0),
            pl.BlockSpec((D, B), c0),
            pl.BlockSpec((D, B), c0),
            pl.BlockSpec((RB, D, B), blk),
            pl.BlockSpec((RB, D, B), blk),
        ],
        out_specs=[
            pl.BlockSpec((RB, D, B), blk),
            pl.BlockSpec((RB, D, B), blk),
        ],
        out_shape=[
            jax.ShapeDtypeStruct((NU, D, B), jnp.float32),
            jax.ShapeDtypeStruct((NI, D, B), jnp.float32),
        ],
    )(ids_u, ids_i, emb_u_t, emb_i_t, t_u, t_i)


def kernel(user_ids, item_ids, user_features, item_features,
           user_memory, item_memory,
           w_ih_u, w_hh_u, b_ih_u, b_hh_u,
           w_ih_i, w_hh_i, b_ih_i, b_hh_i):
    uids = user_ids.astype(jnp.int32)
    iids = item_ids.astype(jnp.int32)
    ids_u = uids.reshape(1, B)
    ids_i = iids.reshape(1, B)

    t_u = jnp.transpose(user_memory, (1, 2, 0))
    t_i = jnp.transpose(item_memory, (1, 2, 0))
    t2_u = t_u.reshape(NU * D * B // 128, 128)
    t2_i = t_i.reshape(NI * D * B // 128, 128)
    uf_t = jnp.transpose(user_features)
    if_t = jnp.transpose(item_features)

    bidx = jnp.arange(B, dtype=jnp.int32)
    drows = 4 * jnp.arange(D, dtype=jnp.int32)
    idx_u = ((uids * (D * B // 128) + bidx // 128)[:, None]
             + drows[None, :]).reshape(-1)
    idx_i = ((iids * (D * B // 128) + bidx // 128)[:, None]
             + drows[None, :]).reshape(-1)

    ue_bm, ie_bm = _sc_gather(t2_u, t2_i, idx_u, idx_i)

    def wsplit(w):
        return (w[:, :D], w[:, D:D + FU], w[:, D + FU:D + FU + D],
                w[:, D + FU + D:])

    new_ue_t, new_ie_t = _gru(
        ue_bm, ie_bm, uf_t, if_t,
        wsplit(w_ih_u), b_ih_u.reshape(3 * H, 1), b_hh_u.reshape(3 * H, 1),
        wsplit(w_ih_i), b_ih_i.reshape(3 * H, 1), b_hh_i.reshape(3 * H, 1))

    new_t_u, new_t_i = _copy_scatter(t_u, t_i, new_ue_t, new_ie_t,
                                     ids_u, ids_i)

    return (jnp.transpose(new_ue_t),
            jnp.transpose(new_ie_t),
            jnp.transpose(new_t_u, (2, 0, 1)),
            jnp.transpose(new_t_i, (2, 0, 1)))


# speedup vs baseline: 2.0577x; 2.0577x over previous
import jax
import jax.numpy as jnp
from jax import lax
from jax.experimental import pallas as pl
from jax.experimental.pallas import tpu as pltpu

B = 512
NU = 1000
NI = 1000
D = 64
H = D
FU = 4
FI = 4
IN = D + FU + D + FI

RB = 50


def _gru_compute(ue_v, ie_v, uf_v, if_v, wih, bih, bhh):
    x = jnp.concatenate([ue_v, uf_v, ie_v, if_v], axis=0)
    gi = lax.dot_general(wih, x, (((1,), (0,)), ((), ())),
                         preferred_element_type=jnp.float32)
    gi = gi + bih
    r = jax.nn.sigmoid(gi[:H] + bhh[:H])
    z = jax.nn.sigmoid(gi[H:2 * H] + bhh[H:2 * H])
    n = jnp.tanh(gi[2 * H:] + r * bhh[2 * H:])
    h = (1.0 - z) * n
    nrm = jnp.sqrt(jnp.sum(h * h, axis=0, keepdims=True))
    return h / jnp.maximum(nrm, 1e-12)


GB = 32


def _gather_gru_body(ru_ref, ri_ref, ids_u_ref, ids_i_ref, *rest):
    slabs_u = rest[:GB]
    slabs_i = rest[GB:2 * GB]
    (wih_u, bih_u, bhh_u, wih_i, bih_i, bhh_i, uf_ref, if_ref,
     uo_ref, io_ref, acc_u, acc_i) = rest[2 * GB:]
    step = pl.program_id(0)
    nsteps = pl.num_programs(0)

    @pl.when(step == 0)
    def _():
        acc_u[...] = jnp.zeros_like(acc_u)
        acc_i[...] = jnp.zeros_like(acc_i)

    ids_u = ids_u_ref[...]
    ids_i = ids_i_ref[...]
    for rows_ref, ids, slabs, acc in ((ru_ref, ids_u, slabs_u, acc_u),
                                      (ri_ref, ids_i, slabs_i, acc_i)):
        masks = [ids == rows_ref[j * nsteps + step] for j in range(GB)]
        cand = slabs[GB - 1][0]
        for j in range(GB - 2, -1, -1):
            cand = jnp.where(masks[j], slabs[j][0], cand)
        m_any = masks[0]
        for j in range(1, GB):
            m_any = m_any | masks[j]
        acc[...] = jnp.where(m_any, cand, acc[...])

    @pl.when(step == nsteps - 1)
    def _():
        ue_v, ie_v = acc_u[...], acc_i[...]
        uf_v, if_v = uf_ref[...], if_ref[...]
        uo_ref[...] = _gru_compute(ue_v, ie_v, uf_v, if_v,
                                   wih_u[...], bih_u[...], bhh_u[...])
        io_ref[...] = _gru_compute(ie_v, ue_v, if_v, uf_v,
                                   wih_i[...], bih_i[...], bhh_i[...])


def _gather_gru(t_u, t_i, rows_u, rows_i, ids_u, ids_i, uf_t, if_t,
                wih_u, bih_u, bhh_u, wih_i, bih_i, bhh_i):
    c0 = lambda i, ru, ri: (0, 0)

    steps = B // GB

    def slab_spec(rows_pos, j):
        if rows_pos == 0:
            return pl.BlockSpec(
                (1, D, B), lambda i, ru, ri, j=j: (ru[j * steps + i], 0, 0))
        return pl.BlockSpec(
            (1, D, B), lambda i, ru, ri, j=j: (ri[j * steps + i], 0, 0))

    grid_spec = pltpu.PrefetchScalarGridSpec(
        num_scalar_prefetch=2,
        grid=(B // GB,),
        in_specs=[
            pl.BlockSpec((1, B), c0),
            pl.BlockSpec((1, B), c0),
            *[slab_spec(0, j) for j in range(GB)],
            *[slab_spec(1, j) for j in range(GB)],
            pl.BlockSpec((3 * H, IN), c0),
            pl.BlockSpec((3 * H, 1), c0),
            pl.BlockSpec((3 * H, 1), c0),
            pl.BlockSpec((3 * H, IN), c0),
            pl.BlockSpec((3 * H, 1), c0),
            pl.BlockSpec((3 * H, 1), c0),
            pl.BlockSpec((FU, B), c0),
            pl.BlockSpec((FI, B), c0),
        ],
        out_specs=[
            pl.BlockSpec((D, B), c0),
            pl.BlockSpec((D, B), c0),
        ],
        scratch_shapes=[
            pltpu.VMEM((D, B), jnp.float32),
            pltpu.VMEM((D, B), jnp.float32),
        ],
    )
    return pl.pallas_call(
        _gather_gru_body,
        grid_spec=grid_spec,
        out_shape=[
            jax.ShapeDtypeStruct((D, B), jnp.float32),
            jax.ShapeDtypeStruct((D, B), jnp.float32),
        ],
    )(rows_u, rows_i, ids_u, ids_i,
      *([t_u] * GB), *([t_i] * GB),
      wih_u, bih_u, bhh_u, wih_i, bih_i, bhh_i, uf_t, if_t)


def _scatter_body(ids_u_ref, ids_i_ref, emb_u_ref, emb_i_ref,
                  tu_ref, ti_ref, uo_ref, io_ref):
    r0 = pl.program_id(0) * RB
    ids_u = ids_u_ref[...]
    ids_i = ids_i_ref[...]
    emb_u = emb_u_ref[...]
    emb_i = emb_i_ref[...]
    for rr in range(RB):
        uo_ref[rr] = jnp.where(ids_u == r0 + rr, emb_u, tu_ref[rr])
        io_ref[rr] = jnp.where(ids_i == r0 + rr, emb_i, ti_ref[rr])


def _copy_scatter(t_u, t_i, emb_u_t, emb_i_t, ids_u, ids_i):
    c0 = lambda i: (0, 0)
    blk = lambda i: (i, 0, 0)
    return pl.pallas_call(
        _scatter_body,
        grid=(NU // RB,),
        in_specs=[
            pl.BlockSpec((1, B), c0),
            pl.BlockSpec((1, B), c0),
            pl.BlockSpec((D, B), c0),
            pl.BlockSpec((D, B), c0),
            pl.BlockSpec((RB, D, B), blk),
            pl.BlockSpec((RB, D, B), blk),
        ],
        out_specs=[
            pl.BlockSpec((RB, D, B), blk),
            pl.BlockSpec((RB, D, B), blk),
        ],
        out_shape=[
            jax.ShapeDtypeStruct((NU, D, B), jnp.float32),
            jax.ShapeDtypeStruct((NI, D, B), jnp.float32),
        ],
    )(ids_u, ids_i, emb_u_t, emb_i_t, t_u, t_i)


def kernel(user_ids, item_ids, user_features, item_features,
           user_memory, item_memory,
           w_ih_u, w_hh_u, b_ih_u, b_hh_u,
           w_ih_i, w_hh_i, b_ih_i, b_hh_i):
    ids_u = user_ids.astype(jnp.int32).reshape(1, B)
    ids_i = item_ids.astype(jnp.int32).reshape(1, B)

    t_u = jnp.transpose(user_memory, (1, 2, 0))
    t_i = jnp.transpose(item_memory, (1, 2, 0))
    uf_t = jnp.transpose(user_features)
    if_t = jnp.transpose(item_features)

    rows_u = jnp.sort(user_ids.astype(jnp.int32))
    rows_i = jnp.sort(item_ids.astype(jnp.int32))

    new_ue_t, new_ie_t = _gather_gru(
        t_u, t_i, rows_u, rows_i, ids_u, ids_i, uf_t, if_t,
        w_ih_u, b_ih_u.reshape(3 * H, 1), b_hh_u.reshape(3 * H, 1),
        w_ih_i, b_ih_i.reshape(3 * H, 1), b_hh_i.reshape(3 * H, 1))

    new_t_u, new_t_i = _copy_scatter(t_u, t_i, new_ue_t, new_ie_t,
                                     ids_u, ids_i)

    return (jnp.transpose(new_ue_t),
            jnp.transpose(new_ie_t),
            jnp.transpose(new_t_u, (2, 0, 1)),
            jnp.transpose(new_t_i, (2, 0, 1)))
